# baseline (speedup 1.0000x reference)
import jax
import jax.numpy as jnp
from jax.experimental import pallas as pl
from jax.experimental.pallas import tpu as pltpu

_BM = 400
_NBUF = 2
_XCHUNK = 1000


def _fused_kernel(wt_ref, b_ref, x_hbm, adj_hbm, out_hbm, xw_ref, xbuf_ref, xsem):
    n = adj_hbm.shape[0]
    d_out = out_hbm.shape[1]
    n_chunks = n // _XCHUNK

    def _xcopy(c):
        return pltpu.make_async_copy(
            x_hbm.at[pl.ds(c * _XCHUNK, _XCHUNK), :],
            xbuf_ref.at[c % 2],
            xsem.at[c % 2],
        )

    _xcopy(0).start()

    def _agg_body(adj_blk_ref, out_blk_ref):
        @pl.when(pl.program_id(0) == 0)
        def _stage1():
            _xcopy(1).start()
            for c in range(n_chunks):
                _xcopy(c).wait()
                xw = jnp.dot(
                    xbuf_ref[c % 2].astype(jnp.bfloat16),
                    wt_ref[...],
                    preferred_element_type=jnp.float32,
                )
                xw_ref[pl.ds(c * _XCHUNK, _XCHUNK), :] = (
                    xw + b_ref[...]
                ).astype(jnp.bfloat16)
                if c + 2 < n_chunks:
                    _xcopy(c + 2).start()

        out_blk_ref[...] = jnp.dot(
            adj_blk_ref[...].astype(jnp.bfloat16),
            xw_ref[...],
            preferred_element_type=jnp.float32,
        )

    pipeline = pltpu.emit_pipeline(
        _agg_body,
        grid=(n // _BM,),
        in_specs=[
            pl.BlockSpec(
                (_BM, n),
                lambda i: (i, 0),
                pipeline_mode=pl.Buffered(buffer_count=_NBUF),
            )
        ],
        out_specs=[pl.BlockSpec((_BM, d_out), lambda i: (i, 0))],
    )
    pipeline(adj_hbm, out_hbm)


def kernel(x, adj, W, b):
    n, d_in = x.shape
    d_out = W.shape[0]
    wt_bf = W.T.astype(jnp.bfloat16)
    b2d = b.reshape(1, d_out)

    out = pl.pallas_call(
        _fused_kernel,
        in_specs=[
            pl.BlockSpec(memory_space=pltpu.VMEM),
            pl.BlockSpec(memory_space=pltpu.VMEM),
            pl.BlockSpec(memory_space=pl.ANY),
            pl.BlockSpec(memory_space=pl.ANY),
        ],
        out_specs=pl.BlockSpec(memory_space=pl.ANY),
        out_shape=jax.ShapeDtypeStruct((n, d_out), jnp.float32),
        scratch_shapes=[
            pltpu.VMEM((n, d_out), jnp.bfloat16),
            pltpu.VMEM((2, _XCHUNK, d_in), jnp.float32),
            pltpu.SemaphoreType.DMA((2,)),
        ],
    )(wt_bf, b2d, x, adj)
    return out


# speedup vs baseline: 1.0308x; 1.0308x over previous
import jax
import jax.numpy as jnp
from jax.experimental import pallas as pl
from jax.experimental.pallas import tpu as pltpu

_BM = 200
_NBUF = 5
_XCHUNK = 1000


def _fused_kernel(wt_ref, b_ref, x_hbm, adj_hbm, out_hbm, xw_ref, xbuf_ref, xsem):
    n = adj_hbm.shape[0]
    d_out = out_hbm.shape[1]
    n_chunks = n // _XCHUNK

    def _xcopy(c):
        return pltpu.make_async_copy(
            x_hbm.at[pl.ds(c * _XCHUNK, _XCHUNK), :],
            xbuf_ref.at[c % 2],
            xsem.at[c % 2],
        )

    _xcopy(0).start()

    def _agg_body(adj_blk_ref, out_blk_ref):
        @pl.when(pl.program_id(0) == 0)
        def _stage1():
            _xcopy(1).start()
            for c in range(n_chunks):
                _xcopy(c).wait()
                xw = jnp.dot(
                    xbuf_ref[c % 2].astype(jnp.bfloat16),
                    wt_ref[...],
                    preferred_element_type=jnp.float32,
                )
                xw_ref[pl.ds(c * _XCHUNK, _XCHUNK), :] = (
                    xw + b_ref[...]
                ).astype(jnp.bfloat16)
                if c + 2 < n_chunks:
                    _xcopy(c + 2).start()

        out_blk_ref[...] = jnp.dot(
            adj_blk_ref[...].astype(jnp.bfloat16),
            xw_ref[...],
            preferred_element_type=jnp.float32,
        )

    pipeline = pltpu.emit_pipeline(
        _agg_body,
        grid=(n // _BM,),
        in_specs=[
            pl.BlockSpec(
                (_BM, n),
                lambda i: (i, 0),
                pipeline_mode=pl.Buffered(buffer_count=_NBUF),
            )
        ],
        out_specs=[pl.BlockSpec((_BM, d_out), lambda i: (i, 0))],
    )
    pipeline(adj_hbm, out_hbm)


def kernel(x, adj, W, b):
    n, d_in = x.shape
    d_out = W.shape[0]
    wt_bf = W.T.astype(jnp.bfloat16)
    b2d = b.reshape(1, d_out)

    out = pl.pallas_call(
        _fused_kernel,
        in_specs=[
            pl.BlockSpec(memory_space=pltpu.VMEM),
            pl.BlockSpec(memory_space=pltpu.VMEM),
            pl.BlockSpec(memory_space=pl.ANY),
            pl.BlockSpec(memory_space=pl.ANY),
        ],
        out_specs=pl.BlockSpec(memory_space=pl.ANY),
        out_shape=jax.ShapeDtypeStruct((n, d_out), jnp.float32),
        scratch_shapes=[
            pltpu.VMEM((n, d_out), jnp.bfloat16),
            pltpu.VMEM((2, _XCHUNK, d_in), jnp.float32),
            pltpu.SemaphoreType.DMA((2,)),
        ],
    )(wt_bf, b2d, x, adj)
    return out


# speedup vs baseline: 1.0484x; 1.0170x over previous
import jax
import jax.numpy as jnp
from jax.experimental import pallas as pl
from jax.experimental.pallas import tpu as pltpu

_BM = 200
_NBUF = 4
_XCHUNK = 1000


def _fused_kernel(wt_ref, b_ref, x_hbm, adj_hbm, out_hbm, xw_ref, xbuf_ref, xsem):
    n = adj_hbm.shape[0]
    d_out = out_hbm.shape[1]
    n_chunks = n // _XCHUNK

    def _xcopy(c):
        return pltpu.make_async_copy(
            x_hbm.at[pl.ds(c * _XCHUNK, _XCHUNK), :],
            xbuf_ref.at[c % 2],
            xsem.at[c % 2],
        )

    _xcopy(0).start()

    def _agg_body(adj_blk_ref, out_blk_ref):
        @pl.when(pl.program_id(0) == 0)
        def _stage1():
            _xcopy(1).start()
            for c in range(n_chunks):
                _xcopy(c).wait()
                xw = jnp.dot(
                    xbuf_ref[c % 2].astype(jnp.bfloat16),
                    wt_ref[...],
                    preferred_element_type=jnp.float32,
                )
                xw_ref[pl.ds(c * _XCHUNK, _XCHUNK), :] = (
                    xw + b_ref[...]
                ).astype(jnp.bfloat16)
                if c + 2 < n_chunks:
                    _xcopy(c + 2).start()

        out_blk_ref[...] = jnp.dot(
            adj_blk_ref[...].astype(jnp.bfloat16),
            xw_ref[...],
            preferred_element_type=jnp.float32,
        )

    pipeline = pltpu.emit_pipeline(
        _agg_body,
        grid=(n // _BM,),
        in_specs=[
            pl.BlockSpec(
                (_BM, n),
                lambda i: (i, 0),
                pipeline_mode=pl.Buffered(buffer_count=_NBUF),
            )
        ],
        out_specs=[pl.BlockSpec((_BM, d_out), lambda i: (i, 0))],
    )
    pipeline(adj_hbm, out_hbm)


def kernel(x, adj, W, b):
    n, d_in = x.shape
    d_out = W.shape[0]
    wt_bf = W.T.astype(jnp.bfloat16)
    b2d = b.reshape(1, d_out)

    out = pl.pallas_call(
        _fused_kernel,
        in_specs=[
            pl.BlockSpec(memory_space=pltpu.VMEM),
            pl.BlockSpec(memory_space=pltpu.VMEM),
            pl.BlockSpec(memory_space=pl.ANY),
            pl.BlockSpec(memory_space=pl.ANY),
        ],
        out_specs=pl.BlockSpec(memory_space=pl.ANY),
        out_shape=jax.ShapeDtypeStruct((n, d_out), jnp.float32),
        scratch_shapes=[
            pltpu.VMEM((n, d_out), jnp.bfloat16),
            pltpu.VMEM((2, _XCHUNK, d_in), jnp.float32),
            pltpu.SemaphoreType.DMA((2,)),
        ],
    )(wt_bf, b2d, x, adj)
    return out


# speedup vs baseline: 1.0611x; 1.0121x over previous
import jax
import jax.numpy as jnp
from jax.experimental import pallas as pl
from jax.experimental.pallas import tpu as pltpu

_BM = 200
_NBUF = 4
_XCHUNK = 2000


def _fused_kernel(wt_ref, b_ref, x_hbm, adj_hbm, out_hbm, xw_ref, xbuf_ref, xsem):
    n = adj_hbm.shape[0]
    d_out = out_hbm.shape[1]
    n_chunks = n // _XCHUNK

    def _xcopy(c):
        return pltpu.make_async_copy(
            x_hbm.at[pl.ds(c * _XCHUNK, _XCHUNK), :],
            xbuf_ref.at[c % 2],
            xsem.at[c % 2],
        )

    _xcopy(0).start()

    def _agg_body(adj_blk_ref, out_blk_ref):
        @pl.when(pl.program_id(0) == 0)
        def _stage1():
            _xcopy(1).start()
            for c in range(n_chunks):
                _xcopy(c).wait()
                xw = jnp.dot(
                    xbuf_ref[c % 2].astype(jnp.bfloat16),
                    wt_ref[...],
                    preferred_element_type=jnp.float32,
                )
                xw_ref[pl.ds(c * _XCHUNK, _XCHUNK), :] = (
                    xw + b_ref[...]
                ).astype(jnp.bfloat16)
                if c + 2 < n_chunks:
                    _xcopy(c + 2).start()

        out_blk_ref[...] = jnp.dot(
            adj_blk_ref[...].astype(jnp.bfloat16),
            xw_ref[...],
            preferred_element_type=jnp.float32,
        )

    pipeline = pltpu.emit_pipeline(
        _agg_body,
        grid=(n // _BM,),
        in_specs=[
            pl.BlockSpec(
                (_BM, n),
                lambda i: (i, 0),
                pipeline_mode=pl.Buffered(buffer_count=_NBUF),
            )
        ],
        out_specs=[pl.BlockSpec((_BM, d_out), lambda i: (i, 0))],
    )
    pipeline(adj_hbm, out_hbm)


def kernel(x, adj, W, b):
    n, d_in = x.shape
    d_out = W.shape[0]
    wt_bf = W.T.astype(jnp.bfloat16)
    b2d = b.reshape(1, d_out)

    out = pl.pallas_call(
        _fused_kernel,
        in_specs=[
            pl.BlockSpec(memory_space=pltpu.VMEM),
            pl.BlockSpec(memory_space=pltpu.VMEM),
            pl.BlockSpec(memory_space=pl.ANY),
            pl.BlockSpec(memory_space=pl.ANY),
        ],
        out_specs=pl.BlockSpec(memory_space=pl.ANY),
        out_shape=jax.ShapeDtypeStruct((n, d_out), jnp.float32),
        scratch_shapes=[
            pltpu.VMEM((n, d_out), jnp.bfloat16),
            pltpu.VMEM((2, _XCHUNK, d_in), jnp.float32),
            pltpu.SemaphoreType.DMA((2,)),
        ],
    )(wt_bf, b2d, x, adj)
    return out
